# Initial kernel scaffold; baseline (speedup 1.0000x reference)
import jax
import jax.numpy as jnp
from jax.experimental import pallas as pl


def kernel(movements, utterances, votes, hive_values, locations):
    raise NotImplementedError("write your pallas kernel here")


# speedup vs baseline: 6.0730x; 6.0730x over previous
import functools

import jax
import jax.numpy as jnp
from jax import lax
from jax.experimental import pallas as pl
from jax.experimental.pallas import tpu as pltpu
from jax.experimental.pallas import tpu_sc as plsc

B = 512
NUM_AGENTS = 64
NUM_HIVES = 16
NUM_ENTITIES = 80
LANES = 16
NUM_WORKERS = 32
B_PER_W = B // NUM_WORKERS


def _sc_vote_body(votes_hbm, hv_hbm, mf_hbm, part_hbm,
                  votes_v, hv_v, counts_v, mf_v, part_v):
    c = lax.axis_index("c")
    s = lax.axis_index("s")
    wid = s * 2 + c
    b0 = wid * B_PER_W

    row = NUM_AGENTS * NUM_HIVES
    pltpu.sync_copy(votes_hbm.at[pl.ds(b0 * row, B_PER_W * row)], votes_v)
    pltpu.sync_copy(hv_hbm.at[pl.ds(b0 * NUM_HIVES, B_PER_W * NUM_HIVES)], hv_v)

    lane = lax.iota(jnp.int32, LANES)
    lane_row = lane * row
    lane_hv = lane * NUM_HIVES
    zero16 = jnp.zeros((LANES,), jnp.float32)
    ones16 = jnp.ones((LANES,), jnp.float32)
    for h in range(NUM_HIVES):
        counts_v[pl.ds(h * LANES, LANES)] = zero16

    def agent_body(a, carry):
        base = lane_row + a * NUM_HIVES
        best_val = jnp.full((LANES,), -jnp.inf, jnp.float32)
        best_idx = jnp.zeros((LANES,), jnp.int32)
        for h in range(NUM_HIVES):
            h_splat = jnp.full((LANES,), h, jnp.int32)
            col = plsc.load_gather(votes_v, [base + h])
            m = col > best_val
            best_val = jnp.where(m, col, best_val)
            best_idx = jnp.where(m, h_splat, best_idx)
        plsc.addupdate_scatter(counts_v, [best_idx * LANES + lane], ones16)
        return carry

    lax.fori_loop(0, NUM_AGENTS, agent_body, 0)

    mf = zero16
    val = zero16
    for h in range(NUM_HIVES):
        ch = counts_v[pl.ds(h * LANES, LANES)]
        mf = jnp.maximum(mf, ch)
        hcol = plsc.load_gather(hv_v, [lane_hv + h])
        val = val + ch * hcol

    max_freq = mf * (1.0 / NUM_AGENTS)
    x = 30.0 * (max_freq - 0.7)
    term = val * (1.0 + jnp.exp(x)) * (1.0 / 100.0)

    mf_v[...] = max_freq
    part_v[...] = term
    pltpu.sync_copy(mf_v, mf_hbm.at[pl.ds(b0, B_PER_W)])
    pltpu.sync_copy(part_v, part_hbm.at[wid])


_sc_vote = functools.partial(
    pl.kernel,
    out_type=(jax.ShapeDtypeStruct((B,), jnp.float32),
              jax.ShapeDtypeStruct((NUM_WORKERS, LANES), jnp.float32)),
    mesh=plsc.VectorSubcoreMesh(core_axis_name="c", subcore_axis_name="s"),
    compiler_params=pltpu.CompilerParams(needs_layout_passes=False),
    scratch_types=[
        pltpu.VMEM((B_PER_W * NUM_AGENTS * NUM_HIVES,), jnp.float32),
        pltpu.VMEM((B_PER_W * NUM_HIVES,), jnp.float32),
        pltpu.VMEM((NUM_HIVES * LANES,), jnp.float32),
        pltpu.VMEM((LANES,), jnp.float32),
        pltpu.VMEM((LANES,), jnp.float32),
    ],
)(_sc_vote_body)


def _tc_cost_body(mv_ref, part_ref, out_ref):
    a = mv_ref[...]
    sq = a * a
    i = lax.broadcasted_iota(jnp.int32, (2 * NUM_ENTITIES, NUM_ENTITIES), 0)
    j = lax.broadcasted_iota(jnp.int32, (2 * NUM_ENTITIES, NUM_ENTITIES), 1)
    pair = (i // 2 == j).astype(jnp.float32)
    s2 = jnp.dot(sq, pair, preferred_element_type=jnp.float32)
    total = jnp.sum(jnp.sqrt(s2)) - jnp.sum(part_ref[...])
    out_ref[...] = jnp.reshape(total, (1, 1))


def kernel(movements, utterances, votes, hive_values, locations):
    hv = hive_values.reshape(B * NUM_HIVES)
    max_freq, parts = _sc_vote(votes.reshape(B * NUM_AGENTS * NUM_HIVES), hv)
    mv2d = movements.reshape(B, 2 * NUM_ENTITIES)
    cost = pl.pallas_call(
        _tc_cost_body,
        out_shape=jax.ShapeDtypeStruct((1, 1), jnp.float32),
    )(mv2d, parts)
    return (cost[0, 0], max_freq)
